# Initial kernel scaffold; baseline (speedup 1.0000x reference)
import jax
import jax.numpy as jnp
from jax.experimental import pallas as pl


def kernel(boxes, scores):
    raise NotImplementedError("write your pallas kernel here")


# speedup vs baseline: 62.1268x; 62.1268x over previous
import jax
import jax.numpy as jnp
from jax import lax
from jax.experimental import pallas as pl
from jax.experimental.pallas import tpu as pltpu
from jax.experimental.pallas import tpu_sc as plsc

N = 20000
NPAD = 20480
NTILES = 16
CH = NPAD // NTILES
TOPK = 5000
K5 = 5120
SB = K5 // NTILES
BLK = 160
NBLK = K5 // BLK
KEEP = 750
_BIG = 1 << 30
_EXPB = 0x3F000000
_MANT = 0x007FFFFF

_iota = lambda: lax.iota(jnp.int32, 16)


def BIGc():
    return jnp.int32(_BIG)


def _splat(v):
    return jnp.full((16,), v)


def _iou_sup(bx1, by1, bx2, by2, ba, x1, y1, x2, y2, a):
    xx1 = jnp.maximum(bx1, x1)
    yy1 = jnp.maximum(by1, y1)
    xx2 = jnp.minimum(bx2, x2)
    yy2 = jnp.minimum(by2, y2)
    w = jnp.maximum(jnp.float32(0.0), xx2 - xx1 + jnp.float32(1.0))
    h = jnp.maximum(jnp.float32(0.0), yy2 - yy1 + jnp.float32(1.0))
    inter = w * h
    ovr = inter / (ba + a - inter)
    return ovr > jnp.float32(0.4)


def _body(scores_hbm, bx1_hbm, by1_hbm, bx2_hbm, by2_hbm, out_hbm,
          sp_k0, sp_v0, sp_k1, sp_v1, sp_grid,
          sp_bx1, sp_by1, sp_bx2, sp_by2, sp_ba,
          sp_kx1, sp_ky1, sp_kx2, sp_ky2, sp_ks, sp_kcnt,
          ls, lk, lv, loff2, gvm, lh, lofs, ttot,
          lb1, lb2, lb3, lb4, cx1, cy1, cx2, cy2, ca, cs, cpos, cal, ral,
          pbx1, pby1, pbx2, pby2, pba, pbs, kv, det):
    w = lax.axis_index("s")
    cb = w * CH
    i16 = _iota()

    pltpu.sync_copy(scores_hbm.at[pl.ds(cb, CH)], ls)

    def _mkkey(i, _):
        s = ls[pl.ds(i * 16, 16)]
        b = plsc.bitcast(s, jnp.int32)
        key = jnp.where(s > jnp.float32(0.5), b & jnp.int32(_MANT), jnp.int32(0))
        lk[pl.ds(i * 16, 16)] = key
        lv[pl.ds(i * 16, 16)] = cb + i * 16 + i16
        return 0
    lax.fori_loop(0, CH // 16, _mkkey, 0)

    def _radix_pass(shift, dst_k, dst_v):
        def _z(i, _):
            lh[pl.ds(i * 16, 16)] = jnp.zeros((16,), jnp.int32)
            return 0
        lax.fori_loop(0, 16, _z, 0)

        def _hist(i, _):
            k16 = lk[pl.ds(i * 16, 16)]
            rd = jnp.int32(255) - ((k16 >> shift) & jnp.int32(255))
            rank, last = plsc.scan_count(rd)
            plsc.addupdate_scatter(lh, [rd], rank, mask=last)
            return 0
        lax.fori_loop(0, CH // 16, _hist, 0)

        pltpu.sync_copy(lh, sp_grid.at[w])
        plsc.subcore_barrier()
        pltpu.sync_copy(sp_grid, gvm)

        def _scan_d(dv, _):
            def _acc(t, c):
                tot, myp = c
                g = gvm[t, pl.ds(dv * 16, 16)]
                return (tot + g, myp + jnp.where(t < w, g, jnp.int32(0)))
            tot, myp = lax.fori_loop(
                0, NTILES, _acc,
                (jnp.zeros((16,), jnp.int32), jnp.zeros((16,), jnp.int32)))
            ttot[pl.ds(dv * 16, 16)] = tot
            lofs[pl.ds(dv * 16, 16)] = myp
            return 0
        lax.fori_loop(0, 16, _scan_d, 0)

        def _excl(dv, base):
            t16 = ttot[pl.ds(dv * 16, 16)]
            c = plsc.cumsum(t16)
            lofs[pl.ds(dv * 16, 16)] = lofs[pl.ds(dv * 16, 16)] + base + c - t16
            return base + c[15]
        lax.fori_loop(0, 16, _excl, jnp.int32(0))

        def _rank(i, _):
            k16 = lk[pl.ds(i * 16, 16)]
            rd = jnp.int32(255) - ((k16 >> shift) & jnp.int32(255))
            rank, last = plsc.scan_count(rd)
            base16 = plsc.load_gather(lofs, [rd])
            plsc.addupdate_scatter(lofs, [rd], rank, mask=last)
            loff2[i >> 3, pl.ds((i & 7) * 16, 16)] = base16 + rank - 1
            return 0
        lax.fori_loop(0, CH // 16, _rank, 0)

        def _scat(r, _):
            pltpu.sync_copy(lk.at[pl.ds(r * 128, 128)], dst_k.at[loff2.at[r]])
            pltpu.sync_copy(lv.at[pl.ds(r * 128, 128)], dst_v.at[loff2.at[r]])
            return 0
        lax.fori_loop(0, CH // 128, _scat, 0)
        plsc.subcore_barrier()

    _radix_pass(jnp.int32(0), sp_k0, sp_v0)
    pltpu.sync_copy(sp_k0.at[pl.ds(cb, CH)], lk)
    pltpu.sync_copy(sp_v0.at[pl.ds(cb, CH)], lv)
    _radix_pass(jnp.int32(8), sp_k1, sp_v1)
    pltpu.sync_copy(sp_k1.at[pl.ds(cb, CH)], lk)
    pltpu.sync_copy(sp_v1.at[pl.ds(cb, CH)], lv)
    _radix_pass(jnp.int32(16), sp_k0, sp_v0)

    sb = w * SB
    pltpu.sync_copy(sp_k0.at[pl.ds(sb, SB)], lk.at[pl.ds(0, SB)])
    pltpu.sync_copy(sp_v0.at[pl.ds(sb, SB)], lv.at[pl.ds(0, SB)])

    pltpu.sync_copy(bx1_hbm, lb1)
    pltpu.sync_copy(by1_hbm, lb2)
    pltpu.sync_copy(bx2_hbm, lb3)
    pltpu.sync_copy(by2_hbm, lb4)

    def _prep(i, _):
        k16 = lk[pl.ds(i * 16, 16)]
        gpos = sb + i * 16 + i16
        val = (k16 > jnp.int32(0)) & (gpos < jnp.int32(TOPK))
        sc = plsc.bitcast(k16 | jnp.int32(_EXPB), jnp.float32)
        idx = lv[pl.ds(i * 16, 16)]
        x1 = plsc.load_gather(lb1, [idx])
        y1 = plsc.load_gather(lb2, [idx])
        x2 = plsc.load_gather(lb3, [idx])
        y2 = plsc.load_gather(lb4, [idx])
        cx1[pl.ds(i * 16, 16)] = x1
        cy1[pl.ds(i * 16, 16)] = y1
        cx2[pl.ds(i * 16, 16)] = x2
        cy2[pl.ds(i * 16, 16)] = y2
        ca[pl.ds(i * 16, 16)] = (x2 - x1 + jnp.float32(1.0)) * (
            y2 - y1 + jnp.float32(1.0))
        cs[pl.ds(i * 16, 16)] = sc
        cal[pl.ds(i * 16, 16)] = val.astype(jnp.int32)
        cpos[pl.ds(i * 16, 16)] = gpos
        return 0
    lax.fori_loop(0, SB // 16, _prep, 0)
    cpos[pl.ds(SB, 16)] = _splat(BIGc())
    cal[pl.ds(SB, 16)] = jnp.zeros((16,), jnp.int32)
    ral[pl.ds(SB, 16)] = jnp.zeros((16,), jnp.int32)

    def _cmp0(v, wp):
        m = cal[pl.ds(v * 16, 16)] != 0
        mi = m.astype(jnp.int32)
        pos = wp + plsc.cumsum(mi) - mi
        plsc.store_scatter(cx1, [pos], cx1[pl.ds(v * 16, 16)], mask=m)
        plsc.store_scatter(cy1, [pos], cy1[pl.ds(v * 16, 16)], mask=m)
        plsc.store_scatter(cx2, [pos], cx2[pl.ds(v * 16, 16)], mask=m)
        plsc.store_scatter(cy2, [pos], cy2[pl.ds(v * 16, 16)], mask=m)
        plsc.store_scatter(ca, [pos], ca[pl.ds(v * 16, 16)], mask=m)
        plsc.store_scatter(cs, [pos], cs[pl.ds(v * 16, 16)], mask=m)
        plsc.store_scatter(cpos, [pos], cpos[pl.ds(v * 16, 16)], mask=m)
        return wp + jnp.sum(mi)
    mc0 = lax.fori_loop(0, SB // 16, _cmp0, jnp.int32(0))

    def _wipe0(v, _):
        l16 = v * 16 + i16
        live = l16 < mc0
        cal[pl.ds(v * 16, 16)] = live.astype(jnp.int32)
        cpos[pl.ds(v * 16, 16)] = jnp.where(live, cpos[pl.ds(v * 16, 16)], BIGc())
        return 0
    lax.fori_loop(0, SB // 16, _wipe0, 0)

    @pl.when(w == 0)
    def _():
        kv[pl.ds(0, 16)] = jnp.zeros((16,), jnp.int32)
        pltpu.sync_copy(kv.at[pl.ds(0, 16)], sp_kcnt.at[pl.ds(0, 16)])
        pltpu.sync_copy(kv.at[pl.ds(0, 16)], sp_kcnt.at[pl.ds(16, 16)])
    plsc.subcore_barrier()

    def _recount():
        def _c(v, n):
            return n + jnp.sum(cal[pl.ds(v * 16, 16)])
        return lax.fori_loop(0, SB // 16, _c, jnp.int32(0))

    def _block(b, mc):
        hi = (b + 1) * BLK

        def _cnt(v, n):
            c16 = cpos[pl.ds(v * 16, 16)]
            return n + jnp.sum((c16 < hi).astype(jnp.int32))
        nb = lax.fori_loop(0, SB // 16, _cnt, jnp.int32(0))

        @pl.when(nb > 0)
        def _resolve():
            def _ral(v, _):
                l16 = v * 16 + i16
                ral[pl.ds(v * 16, 16)] = (
                    (l16 < nb).astype(jnp.int32) * cal[pl.ds(v * 16, 16)])
                return 0
            lax.fori_loop(0, SB // 16 + 1, _ral, 0)

            def _piv(i, _):
                rv = ral[pl.ds(i, 16)][0]

                @pl.when(rv != 0)
                def _():
                    bx1 = _splat(cx1[pl.ds(i, 16)][0])
                    by1 = _splat(cy1[pl.ds(i, 16)][0])
                    bx2 = _splat(cx2[pl.ds(i, 16)][0])
                    by2 = _splat(cy2[pl.ds(i, 16)][0])
                    ba = _splat(ca[pl.ds(i, 16)][0])

                    def _sup(v, _):
                        jl = v * 16 + i16
                        r16 = ral[pl.ds(v * 16, 16)]
                        sup = _iou_sup(bx1, by1, bx2, by2, ba,
                                       cx1[pl.ds(v * 16, 16)],
                                       cy1[pl.ds(v * 16, 16)],
                                       cx2[pl.ds(v * 16, 16)],
                                       cy2[pl.ds(v * 16, 16)],
                                       ca[pl.ds(v * 16, 16)])
                        kill = sup & (jl > i)
                        ral[pl.ds(v * 16, 16)] = jnp.where(
                            kill, jnp.int32(0), r16)
                        return 0
                    lax.fori_loop(i >> 4, (nb + 15) >> 4, _sup, 0)
                return 0
            lax.fori_loop(0, nb, _piv, 0)

            def _cmp(v, wp):
                m = ral[pl.ds(v * 16, 16)] != 0
                mi = m.astype(jnp.int32)
                pos = wp + plsc.cumsum(mi) - mi
                plsc.store_scatter(pbx1, [pos], cx1[pl.ds(v * 16, 16)], mask=m)
                plsc.store_scatter(pby1, [pos], cy1[pl.ds(v * 16, 16)], mask=m)
                plsc.store_scatter(pbx2, [pos], cx2[pl.ds(v * 16, 16)], mask=m)
                plsc.store_scatter(pby2, [pos], cy2[pl.ds(v * 16, 16)], mask=m)
                plsc.store_scatter(pba, [pos], ca[pl.ds(v * 16, 16)], mask=m)
                plsc.store_scatter(pbs, [pos], cs[pl.ds(v * 16, 16)], mask=m)
                return wp + jnp.sum(mi)
            kb = lax.fori_loop(0, BLK // 16, _cmp, jnp.int32(0))

            pltpu.sync_copy(pbx1.at[pl.ds(0, BLK)], sp_bx1)
            pltpu.sync_copy(pby1.at[pl.ds(0, BLK)], sp_by1)
            pltpu.sync_copy(pbx2.at[pl.ds(0, BLK)], sp_bx2)
            pltpu.sync_copy(pby2.at[pl.ds(0, BLK)], sp_by2)
            pltpu.sync_copy(pba.at[pl.ds(0, BLK)], sp_ba)
            pltpu.sync_copy(pbx1.at[pl.ds(0, BLK)], sp_kx1.at[pl.ds(b * BLK, BLK)])
            pltpu.sync_copy(pby1.at[pl.ds(0, BLK)], sp_ky1.at[pl.ds(b * BLK, BLK)])
            pltpu.sync_copy(pbx2.at[pl.ds(0, BLK)], sp_kx2.at[pl.ds(b * BLK, BLK)])
            pltpu.sync_copy(pby2.at[pl.ds(0, BLK)], sp_ky2.at[pl.ds(b * BLK, BLK)])
            pltpu.sync_copy(pbs.at[pl.ds(0, BLK)], sp_ks.at[pl.ds(b * BLK, BLK)])
            ch = (b >> 4) * 16
            pltpu.sync_copy(sp_kcnt.at[pl.ds(ch, 16)], kv.at[pl.ds(0, 16)])
            kv16 = kv[pl.ds(0, 16)]
            kv[pl.ds(0, 16)] = jnp.where(i16 == b - ch, kb, kv16)
            pltpu.sync_copy(kv.at[pl.ds(0, 16)], sp_kcnt.at[pl.ds(ch, 16)])

            def _shift(v, _):
                src = nb + v * 16 + i16
                msk = src < mc
                cx1[pl.ds(v * 16, 16)] = plsc.load_gather(cx1, [src], mask=msk)
                cy1[pl.ds(v * 16, 16)] = plsc.load_gather(cy1, [src], mask=msk)
                cx2[pl.ds(v * 16, 16)] = plsc.load_gather(cx2, [src], mask=msk)
                cy2[pl.ds(v * 16, 16)] = plsc.load_gather(cy2, [src], mask=msk)
                ca[pl.ds(v * 16, 16)] = plsc.load_gather(ca, [src], mask=msk)
                cs[pl.ds(v * 16, 16)] = plsc.load_gather(cs, [src], mask=msk)
                cal[pl.ds(v * 16, 16)] = jnp.where(
                    msk, plsc.load_gather(cal, [src], mask=msk), jnp.int32(0))
                cpos[pl.ds(v * 16, 16)] = jnp.where(
                    msk, plsc.load_gather(cpos, [src], mask=msk), BIGc())
                return 0
            lax.fori_loop(0, SB // 16, _shift, 0)

        mc = jnp.where(nb > 0, mc - nb, mc)
        plsc.subcore_barrier()

        ch = (b >> 4) * 16
        pltpu.sync_copy(sp_kcnt.at[pl.ds(ch, 16)], kv.at[pl.ds(0, 16)])
        kb_s = kv[pl.ds(0, 16)][_splat(b - ch)][0]

        @pl.when((kb_s > 0) & (mc > 0))
        def _apply():
            pltpu.sync_copy(sp_bx1, pbx1.at[pl.ds(0, BLK)])
            pltpu.sync_copy(sp_by1, pby1.at[pl.ds(0, BLK)])
            pltpu.sync_copy(sp_bx2, pbx2.at[pl.ds(0, BLK)])
            pltpu.sync_copy(sp_by2, pby2.at[pl.ds(0, BLK)])
            pltpu.sync_copy(sp_ba, pba.at[pl.ds(0, BLK)])
            nv = (mc + 15) >> 4

            def _pv(p, _):
                bx1 = _splat(pbx1[pl.ds(p, 16)][0])
                by1 = _splat(pby1[pl.ds(p, 16)][0])
                bx2 = _splat(pbx2[pl.ds(p, 16)][0])
                by2 = _splat(pby2[pl.ds(p, 16)][0])
                ba = _splat(pba[pl.ds(p, 16)][0])

                def _sup(v, _):
                    a16 = cal[pl.ds(v * 16, 16)]
                    sup = _iou_sup(bx1, by1, bx2, by2, ba,
                                   cx1[pl.ds(v * 16, 16)],
                                   cy1[pl.ds(v * 16, 16)],
                                   cx2[pl.ds(v * 16, 16)],
                                   cy2[pl.ds(v * 16, 16)],
                                   ca[pl.ds(v * 16, 16)])
                    cal[pl.ds(v * 16, 16)] = jnp.where(sup, jnp.int32(0), a16)
                    return 0
                lax.fori_loop(0, nv, _sup, 0)
                return 0
            lax.fori_loop(0, kb_s, _pv, 0)

            def _cmp2(v, wp):
                m = cal[pl.ds(v * 16, 16)] != 0
                mi = m.astype(jnp.int32)
                pos = wp + plsc.cumsum(mi) - mi
                plsc.store_scatter(cx1, [pos], cx1[pl.ds(v * 16, 16)], mask=m)
                plsc.store_scatter(cy1, [pos], cy1[pl.ds(v * 16, 16)], mask=m)
                plsc.store_scatter(cx2, [pos], cx2[pl.ds(v * 16, 16)], mask=m)
                plsc.store_scatter(cy2, [pos], cy2[pl.ds(v * 16, 16)], mask=m)
                plsc.store_scatter(ca, [pos], ca[pl.ds(v * 16, 16)], mask=m)
                plsc.store_scatter(cs, [pos], cs[pl.ds(v * 16, 16)], mask=m)
                plsc.store_scatter(cpos, [pos], cpos[pl.ds(v * 16, 16)], mask=m)
                return wp + jnp.sum(mi)
            wp = lax.fori_loop(0, nv, _cmp2, jnp.int32(0))

            def _wipe(v, _):
                l16 = v * 16 + i16
                live = l16 < wp
                cal[pl.ds(v * 16, 16)] = live.astype(jnp.int32)
                cpos[pl.ds(v * 16, 16)] = jnp.where(
                    live, cpos[pl.ds(v * 16, 16)], BIGc())
                return 0
            lax.fori_loop(0, SB // 16, _wipe, 0)
        return _recount()

    mc = lax.fori_loop(0, NBLK, _block, mc0)
    del mc
    plsc.subcore_barrier()

    @pl.when(w == 0)
    def _emit():
        pltpu.sync_copy(sp_kcnt, kv.at[pl.ds(0, 32)])

        def _zero(v, _):
            det[pl.ds(v * 16, 16)] = jnp.zeros((16,), jnp.float32)
            return 0
        lax.fori_loop(0, det.shape[0] // 16, _zero, 0)

        def _blk(b, off):
            chv = (b >> 4) * 16
            kb = kv[pl.ds(chv, 16)][_splat(b - chv)][0]

            @pl.when(kb > 0)
            def _():
                pltpu.sync_copy(sp_kx1.at[pl.ds(b * BLK, BLK)], pbx1.at[pl.ds(0, BLK)])
                pltpu.sync_copy(sp_ky1.at[pl.ds(b * BLK, BLK)], pby1.at[pl.ds(0, BLK)])
                pltpu.sync_copy(sp_kx2.at[pl.ds(b * BLK, BLK)], pbx2.at[pl.ds(0, BLK)])
                pltpu.sync_copy(sp_ky2.at[pl.ds(b * BLK, BLK)], pby2.at[pl.ds(0, BLK)])
                pltpu.sync_copy(sp_ks.at[pl.ds(b * BLK, BLK)], pbs.at[pl.ds(0, BLK)])

                def _row(v, _):
                    l16 = v * 16 + i16
                    ridx = off + l16
                    m = (l16 < kb) & (ridx < jnp.int32(KEEP))
                    base5 = ridx * 5
                    plsc.store_scatter(det, [base5], pbx1[pl.ds(v * 16, 16)], mask=m)
                    plsc.store_scatter(det, [base5 + 1], pby1[pl.ds(v * 16, 16)], mask=m)
                    plsc.store_scatter(det, [base5 + 2], pbx2[pl.ds(v * 16, 16)], mask=m)
                    plsc.store_scatter(det, [base5 + 3], pby2[pl.ds(v * 16, 16)], mask=m)
                    plsc.store_scatter(det, [base5 + 4], pbs[pl.ds(v * 16, 16)], mask=m)
                    return 0
                lax.fori_loop(0, BLK // 16, _row, 0)
            return off + kb
        lax.fori_loop(0, NBLK, _blk, jnp.int32(0))
        pltpu.sync_copy(det.at[pl.ds(0, 3760)], out_hbm)


def kernel(boxes, scores):
    scores_p = jnp.zeros((NPAD,), jnp.float32).at[:N].set(scores)
    boxes_t = jnp.zeros((4, NPAD), jnp.float32).at[:, :N].set(boxes.T)
    b1, b2, b3, b4 = boxes_t[0], boxes_t[1], boxes_t[2], boxes_t[3]

    mesh = plsc.VectorSubcoreMesh(
        core_axis_name="c", subcore_axis_name="s", num_cores=1)

    f = pl.kernel(
        _body,
        out_type=jax.ShapeDtypeStruct((3760,), jnp.float32),
        mesh=mesh,
        compiler_params=pltpu.CompilerParams(needs_layout_passes=False),
        scratch_types=[
            pltpu.VMEM_SHARED((NPAD,), jnp.int32),
            pltpu.VMEM_SHARED((NPAD,), jnp.int32),
            pltpu.VMEM_SHARED((NPAD,), jnp.int32),
            pltpu.VMEM_SHARED((NPAD,), jnp.int32),
            pltpu.VMEM_SHARED((NTILES, 256), jnp.int32),
            pltpu.VMEM_SHARED((BLK,), jnp.float32),
            pltpu.VMEM_SHARED((BLK,), jnp.float32),
            pltpu.VMEM_SHARED((BLK,), jnp.float32),
            pltpu.VMEM_SHARED((BLK,), jnp.float32),
            pltpu.VMEM_SHARED((BLK,), jnp.float32),
            pltpu.VMEM_SHARED((K5,), jnp.float32),
            pltpu.VMEM_SHARED((K5,), jnp.float32),
            pltpu.VMEM_SHARED((K5,), jnp.float32),
            pltpu.VMEM_SHARED((K5,), jnp.float32),
            pltpu.VMEM_SHARED((K5,), jnp.float32),
            pltpu.VMEM_SHARED((32,), jnp.int32),
            pltpu.VMEM((CH,), jnp.float32),
            pltpu.VMEM((CH,), jnp.int32),
            pltpu.VMEM((CH,), jnp.int32),
            pltpu.VMEM((CH // 128, 128), jnp.int32),
            pltpu.VMEM((NTILES, 256), jnp.int32),
            pltpu.VMEM((256,), jnp.int32),
            pltpu.VMEM((256,), jnp.int32),
            pltpu.VMEM((256,), jnp.int32),
            pltpu.VMEM((NPAD,), jnp.float32),
            pltpu.VMEM((NPAD,), jnp.float32),
            pltpu.VMEM((NPAD,), jnp.float32),
            pltpu.VMEM((NPAD,), jnp.float32),
            pltpu.VMEM((SB + 16,), jnp.float32),
            pltpu.VMEM((SB + 16,), jnp.float32),
            pltpu.VMEM((SB + 16,), jnp.float32),
            pltpu.VMEM((SB + 16,), jnp.float32),
            pltpu.VMEM((SB + 16,), jnp.float32),
            pltpu.VMEM((SB + 16,), jnp.float32),
            pltpu.VMEM((SB + 16,), jnp.int32),
            pltpu.VMEM((SB + 16,), jnp.int32),
            pltpu.VMEM((SB + 16,), jnp.int32),
            pltpu.VMEM((BLK + 16,), jnp.float32),
            pltpu.VMEM((BLK + 16,), jnp.float32),
            pltpu.VMEM((BLK + 16,), jnp.float32),
            pltpu.VMEM((BLK + 16,), jnp.float32),
            pltpu.VMEM((BLK + 16,), jnp.float32),
            pltpu.VMEM((BLK + 16,), jnp.float32),
            pltpu.VMEM((48,), jnp.int32),
            pltpu.VMEM((3760,), jnp.float32),
        ],
    )
    out = f(scores_p, b1, b2, b3, b4)
    return out[:3750].reshape(KEEP, 5)


# speedup vs baseline: 98.9041x; 1.5920x over previous
import jax
import jax.numpy as jnp
from jax import lax
from jax.experimental import pallas as pl
from jax.experimental.pallas import tpu as pltpu
from jax.experimental.pallas import tpu_sc as plsc

N = 20000
NPAD = 20480
NTILES = 16
CH = NPAD // NTILES
TOPK = 5000
K5 = 5120
SB = K5 // NTILES
BLK = 160
NBLK = K5 // BLK
KEEP = 750
_BIG = 1 << 30
_EXPB = 0x3F000000
_MANT = 0x007FFFFF

_iota = lambda: lax.iota(jnp.int32, 16)


def BIGc():
    return jnp.int32(_BIG)


def _splat(v):
    return jnp.full((16,), v)


def _iou_sup(bx1, by1, bx2, by2, ba, x1, y1, x2, y2, a):
    xx1 = jnp.maximum(bx1, x1)
    yy1 = jnp.maximum(by1, y1)
    xx2 = jnp.minimum(bx2, x2)
    yy2 = jnp.minimum(by2, y2)
    w = jnp.maximum(jnp.float32(0.0), xx2 - xx1 + jnp.float32(1.0))
    h = jnp.maximum(jnp.float32(0.0), yy2 - yy1 + jnp.float32(1.0))
    inter = w * h
    ovr = inter / (ba + a - inter)
    return ovr > jnp.float32(0.4)


def _body(scores_hbm, bx1_hbm, by1_hbm, bx2_hbm, by2_hbm, out_hbm,
          sp_k0, sp_v0, sp_k1, sp_v1, sp_grid,
          sp_blk,
          sp_kx1, sp_ky1, sp_kx2, sp_ky2, sp_ks, sp_kcnt,
          ls, lk, lv, loff2, gvm, lh, lofs, ttot,
          lb1, lb2, lb3, lb4, cx1, cy1, cx2, cy2, ca, cs, cpos, cal, ral,
          lpv, pv, pbs, kv, det):
    w = lax.axis_index("s")
    cb = w * CH
    i16 = _iota()

    pltpu.sync_copy(scores_hbm.at[pl.ds(cb, CH)], ls)

    def _mkkey(i, _):
        s = ls[pl.ds(i * 16, 16)]
        b = plsc.bitcast(s, jnp.int32)
        key = jnp.where(s > jnp.float32(0.5), b & jnp.int32(_MANT), jnp.int32(0))
        lk[pl.ds(i * 16, 16)] = key
        lv[pl.ds(i * 16, 16)] = cb + i * 16 + i16
        return 0
    lax.fori_loop(0, CH // 16, _mkkey, 0)

    def _radix_pass(shift, dst_k, dst_v):
        def _z(i, _):
            lh[pl.ds(i * 16, 16)] = jnp.zeros((16,), jnp.int32)
            return 0
        lax.fori_loop(0, 16, _z, 0)

        def _hist(i, _):
            k16 = lk[pl.ds(i * 16, 16)]
            rd = jnp.int32(255) - ((k16 >> shift) & jnp.int32(255))
            rank, last = plsc.scan_count(rd)
            plsc.addupdate_scatter(lh, [rd], rank, mask=last)
            return 0
        lax.fori_loop(0, CH // 16, _hist, 0)

        pltpu.sync_copy(lh, sp_grid.at[w])
        plsc.subcore_barrier()
        pltpu.sync_copy(sp_grid, gvm)

        def _scan_d(dv, _):
            def _acc(t, c):
                tot, myp = c
                g = gvm[t, pl.ds(dv * 16, 16)]
                return (tot + g, myp + jnp.where(t < w, g, jnp.int32(0)))
            tot, myp = lax.fori_loop(
                0, NTILES, _acc,
                (jnp.zeros((16,), jnp.int32), jnp.zeros((16,), jnp.int32)))
            ttot[pl.ds(dv * 16, 16)] = tot
            lofs[pl.ds(dv * 16, 16)] = myp
            return 0
        lax.fori_loop(0, 16, _scan_d, 0)

        def _excl(dv, base):
            t16 = ttot[pl.ds(dv * 16, 16)]
            c = plsc.cumsum(t16)
            lofs[pl.ds(dv * 16, 16)] = lofs[pl.ds(dv * 16, 16)] + base + c - t16
            return base + c[15]
        lax.fori_loop(0, 16, _excl, jnp.int32(0))

        def _rank(i, _):
            k16 = lk[pl.ds(i * 16, 16)]
            rd = jnp.int32(255) - ((k16 >> shift) & jnp.int32(255))
            rank, last = plsc.scan_count(rd)
            base16 = plsc.load_gather(lofs, [rd])
            plsc.addupdate_scatter(lofs, [rd], rank, mask=last)
            loff2[i >> 3, pl.ds((i & 7) * 16, 16)] = base16 + rank - 1
            return 0
        lax.fori_loop(0, CH // 16, _rank, 0)

        def _scat(r, _):
            pltpu.sync_copy(lk.at[pl.ds(r * 128, 128)], dst_k.at[loff2.at[r]])
            pltpu.sync_copy(lv.at[pl.ds(r * 128, 128)], dst_v.at[loff2.at[r]])
            return 0
        lax.fori_loop(0, CH // 128, _scat, 0)
        plsc.subcore_barrier()

    _radix_pass(jnp.int32(0), sp_k0, sp_v0)
    pltpu.sync_copy(sp_k0.at[pl.ds(cb, CH)], lk)
    pltpu.sync_copy(sp_v0.at[pl.ds(cb, CH)], lv)
    _radix_pass(jnp.int32(8), sp_k1, sp_v1)
    pltpu.sync_copy(sp_k1.at[pl.ds(cb, CH)], lk)
    pltpu.sync_copy(sp_v1.at[pl.ds(cb, CH)], lv)
    _radix_pass(jnp.int32(16), sp_k0, sp_v0)

    sb = w * SB
    pltpu.sync_copy(sp_k0.at[pl.ds(sb, SB)], lk.at[pl.ds(0, SB)])
    pltpu.sync_copy(sp_v0.at[pl.ds(sb, SB)], lv.at[pl.ds(0, SB)])

    pltpu.sync_copy(bx1_hbm, lb1)
    pltpu.sync_copy(by1_hbm, lb2)
    pltpu.sync_copy(bx2_hbm, lb3)
    pltpu.sync_copy(by2_hbm, lb4)

    def _prep(i, _):
        k16 = lk[pl.ds(i * 16, 16)]
        gpos = sb + i * 16 + i16
        val = (k16 > jnp.int32(0)) & (gpos < jnp.int32(TOPK))
        sc = plsc.bitcast(k16 | jnp.int32(_EXPB), jnp.float32)
        idx = lv[pl.ds(i * 16, 16)]
        x1 = plsc.load_gather(lb1, [idx])
        y1 = plsc.load_gather(lb2, [idx])
        x2 = plsc.load_gather(lb3, [idx])
        y2 = plsc.load_gather(lb4, [idx])
        cx1[pl.ds(i * 16, 16)] = x1
        cy1[pl.ds(i * 16, 16)] = y1
        cx2[pl.ds(i * 16, 16)] = x2
        cy2[pl.ds(i * 16, 16)] = y2
        ca[pl.ds(i * 16, 16)] = (x2 - x1 + jnp.float32(1.0)) * (
            y2 - y1 + jnp.float32(1.0))
        cs[pl.ds(i * 16, 16)] = sc
        cal[pl.ds(i * 16, 16)] = val.astype(jnp.int32)
        cpos[pl.ds(i * 16, 16)] = gpos
        return 0
    lax.fori_loop(0, SB // 16, _prep, 0)
    cpos[pl.ds(SB, 16)] = _splat(BIGc())
    cal[pl.ds(SB, 16)] = jnp.zeros((16,), jnp.int32)
    ral[pl.ds(SB, 16)] = jnp.zeros((16,), jnp.int32)

    def _cmp0(v, wp):
        m = cal[pl.ds(v * 16, 16)] != 0
        mi = m.astype(jnp.int32)
        pos = wp + plsc.cumsum(mi) - mi
        plsc.store_scatter(cx1, [pos], cx1[pl.ds(v * 16, 16)], mask=m)
        plsc.store_scatter(cy1, [pos], cy1[pl.ds(v * 16, 16)], mask=m)
        plsc.store_scatter(cx2, [pos], cx2[pl.ds(v * 16, 16)], mask=m)
        plsc.store_scatter(cy2, [pos], cy2[pl.ds(v * 16, 16)], mask=m)
        plsc.store_scatter(ca, [pos], ca[pl.ds(v * 16, 16)], mask=m)
        plsc.store_scatter(cs, [pos], cs[pl.ds(v * 16, 16)], mask=m)
        plsc.store_scatter(cpos, [pos], cpos[pl.ds(v * 16, 16)], mask=m)
        return wp + jnp.sum(mi)
    mc0 = lax.fori_loop(0, SB // 16, _cmp0, jnp.int32(0))

    def _wipe0(v, _):
        l16 = v * 16 + i16
        live = l16 < mc0
        cal[pl.ds(v * 16, 16)] = live.astype(jnp.int32)
        cpos[pl.ds(v * 16, 16)] = jnp.where(live, cpos[pl.ds(v * 16, 16)], BIGc())
        return 0
    lax.fori_loop(0, SB // 16, _wipe0, 0)

    @pl.when(w == 0)
    def _():
        kv[pl.ds(0, 16)] = jnp.zeros((16,), jnp.int32)
        pltpu.sync_copy(kv.at[pl.ds(0, 16)], sp_kcnt.at[pl.ds(0, 16)])
        pltpu.sync_copy(kv.at[pl.ds(0, 16)], sp_kcnt.at[pl.ds(16, 16)])
    plsc.subcore_barrier()

    def _block(b, mc):
        hi = (b + 1) * BLK

        def _cnt(v, n):
            c16 = cpos[pl.ds(v * 16, 16)]
            return n + jnp.sum((c16 < hi).astype(jnp.int32))
        nb = jnp.where(
            w == (b >> 1),
            lax.fori_loop(0, SB // 16, _cnt, jnp.int32(0)), jnp.int32(0))

        @pl.when(nb > 0)
        def _resolve():
            def _ral(v, _):
                l16 = v * 16 + i16
                ral[pl.ds(v * 16, 16)] = (
                    (l16 < nb).astype(jnp.int32) * cal[pl.ds(v * 16, 16)])
                return 0
            lax.fori_loop(0, SB // 16 + 1, _ral, 0)

            def _piv(i, _):
                rv = ral[pl.ds(i, 16)][0]

                @pl.when(rv != 0)
                def _():
                    bx1 = _splat(cx1[pl.ds(i, 16)][0])
                    by1 = _splat(cy1[pl.ds(i, 16)][0])
                    bx2 = _splat(cx2[pl.ds(i, 16)][0])
                    by2 = _splat(cy2[pl.ds(i, 16)][0])
                    ba = _splat(ca[pl.ds(i, 16)][0])

                    def _sup(v, _):
                        jl = v * 16 + i16
                        r16 = ral[pl.ds(v * 16, 16)]
                        sup = _iou_sup(bx1, by1, bx2, by2, ba,
                                       cx1[pl.ds(v * 16, 16)],
                                       cy1[pl.ds(v * 16, 16)],
                                       cx2[pl.ds(v * 16, 16)],
                                       cy2[pl.ds(v * 16, 16)],
                                       ca[pl.ds(v * 16, 16)])
                        kill = sup & (jl > i)
                        ral[pl.ds(v * 16, 16)] = jnp.where(
                            kill, jnp.int32(0), r16)
                        return 0
                    lax.fori_loop(i >> 4, (nb + 15) >> 4, _sup, 0)
                return 0
            lax.fori_loop(0, nb, _piv, 0)

            z16 = _splat(jnp.int32(0))

            def _cmp(v, wp):
                m = ral[pl.ds(v * 16, 16)] != 0
                mi = m.astype(jnp.int32)
                pos = wp + plsc.cumsum(mi) - mi
                plsc.store_scatter(lpv, [z16, pos], cx1[pl.ds(v * 16, 16)], mask=m)
                plsc.store_scatter(lpv, [z16 + 1, pos], cy1[pl.ds(v * 16, 16)], mask=m)
                plsc.store_scatter(lpv, [z16 + 2, pos], cx2[pl.ds(v * 16, 16)], mask=m)
                plsc.store_scatter(lpv, [z16 + 3, pos], cy2[pl.ds(v * 16, 16)], mask=m)
                plsc.store_scatter(lpv, [z16 + 4, pos], ca[pl.ds(v * 16, 16)], mask=m)
                plsc.store_scatter(pbs, [pos], cs[pl.ds(v * 16, 16)], mask=m)
                return wp + jnp.sum(mi)
            kb = lax.fori_loop(0, BLK // 16, _cmp, jnp.int32(0))

            pad_i = kb + i16
            pad_m = i16 < ((-kb) & 3)
            sent = _splat(jnp.float32(-4e6))
            plsc.store_scatter(lpv, [z16, pad_i], sent, mask=pad_m)
            plsc.store_scatter(lpv, [z16 + 1, pad_i], sent, mask=pad_m)
            plsc.store_scatter(lpv, [z16 + 2, pad_i], sent, mask=pad_m)
            plsc.store_scatter(lpv, [z16 + 3, pad_i], sent, mask=pad_m)
            plsc.store_scatter(lpv, [z16 + 4, pad_i], _splat(jnp.float32(1.0)),
                               mask=pad_m)

            pltpu.sync_copy(lpv, sp_blk)
            pltpu.sync_copy(lpv.at[0], sp_kx1.at[pl.ds(b * 256, 256)])
            pltpu.sync_copy(lpv.at[1], sp_ky1.at[pl.ds(b * 256, 256)])
            pltpu.sync_copy(lpv.at[2], sp_kx2.at[pl.ds(b * 256, 256)])
            pltpu.sync_copy(lpv.at[3], sp_ky2.at[pl.ds(b * 256, 256)])
            pltpu.sync_copy(pbs, sp_ks.at[pl.ds(b * 256, 256)])
            ch = (b >> 4) * 16
            pltpu.sync_copy(sp_kcnt.at[pl.ds(ch, 16)], kv.at[pl.ds(0, 16)])
            kv16 = kv[pl.ds(0, 16)]
            kv[pl.ds(0, 16)] = jnp.where(i16 == b - ch, kb, kv16)
            pltpu.sync_copy(kv.at[pl.ds(0, 16)], sp_kcnt.at[pl.ds(ch, 16)])

            def _shift(v, _):
                src = nb + v * 16 + i16
                msk = src < mc
                cx1[pl.ds(v * 16, 16)] = plsc.load_gather(cx1, [src], mask=msk)
                cy1[pl.ds(v * 16, 16)] = plsc.load_gather(cy1, [src], mask=msk)
                cx2[pl.ds(v * 16, 16)] = plsc.load_gather(cx2, [src], mask=msk)
                cy2[pl.ds(v * 16, 16)] = plsc.load_gather(cy2, [src], mask=msk)
                ca[pl.ds(v * 16, 16)] = plsc.load_gather(ca, [src], mask=msk)
                cs[pl.ds(v * 16, 16)] = plsc.load_gather(cs, [src], mask=msk)
                cal[pl.ds(v * 16, 16)] = jnp.where(
                    msk, plsc.load_gather(cal, [src], mask=msk), jnp.int32(0))
                cpos[pl.ds(v * 16, 16)] = jnp.where(
                    msk, plsc.load_gather(cpos, [src], mask=msk), BIGc())
                return 0
            lax.fori_loop(0, SB // 16, _shift, 0)

        mc = jnp.where(nb > 0, mc - nb, mc)
        kv[pl.ds(32, 16)] = _splat(mc)
        plsc.subcore_barrier()

        ch = (b >> 4) * 16
        pltpu.sync_copy(sp_kcnt.at[pl.ds(ch, 16)], kv.at[pl.ds(0, 16)])
        kb_s = kv[pl.ds(0, 16)][_splat(b - ch)][0]

        @pl.when((kb_s > 0) & (mc > 0))
        def _apply():
            pltpu.sync_copy(sp_blk, pv)
            nv = (mc + 15) >> 4
            kb4 = (kb_s + 3) >> 2

            def _pv(p4, _):
                p = p4 * 4

                def _bc(row, off):
                    return _splat(pv[row, pl.ds(p + off, 16)][0])
                ax1, ay1, ax2, ay2, aa = (_bc(0, 0), _bc(1, 0), _bc(2, 0),
                                          _bc(3, 0), _bc(4, 0))
                bx1, by1, bx2, by2, ba = (_bc(0, 1), _bc(1, 1), _bc(2, 1),
                                          _bc(3, 1), _bc(4, 1))
                dx1, dy1, dx2, dy2, da = (_bc(0, 2), _bc(1, 2), _bc(2, 2),
                                          _bc(3, 2), _bc(4, 2))
                ex1, ey1, ex2, ey2, ea = (_bc(0, 3), _bc(1, 3), _bc(2, 3),
                                          _bc(3, 3), _bc(4, 3))

                def _sup(v, _):
                    a16 = cal[pl.ds(v * 16, 16)]
                    x1 = cx1[pl.ds(v * 16, 16)]
                    y1 = cy1[pl.ds(v * 16, 16)]
                    x2 = cx2[pl.ds(v * 16, 16)]
                    y2 = cy2[pl.ds(v * 16, 16)]
                    a = ca[pl.ds(v * 16, 16)]
                    s0 = _iou_sup(ax1, ay1, ax2, ay2, aa, x1, y1, x2, y2, a)
                    s1 = _iou_sup(bx1, by1, bx2, by2, ba, x1, y1, x2, y2, a)
                    s2 = _iou_sup(dx1, dy1, dx2, dy2, da, x1, y1, x2, y2, a)
                    s3 = _iou_sup(ex1, ey1, ex2, ey2, ea, x1, y1, x2, y2, a)
                    sup = (s0 | s1) | (s2 | s3)
                    cal[pl.ds(v * 16, 16)] = jnp.where(sup, jnp.int32(0), a16)
                    return 0
                lax.fori_loop(0, nv, _sup, 0)
                return 0
            lax.fori_loop(0, kb4, _pv, 0)

            def _cmp2(v, wp):
                m = cal[pl.ds(v * 16, 16)] != 0
                mi = m.astype(jnp.int32)
                pos = wp + plsc.cumsum(mi) - mi
                plsc.store_scatter(cx1, [pos], cx1[pl.ds(v * 16, 16)], mask=m)
                plsc.store_scatter(cy1, [pos], cy1[pl.ds(v * 16, 16)], mask=m)
                plsc.store_scatter(cx2, [pos], cx2[pl.ds(v * 16, 16)], mask=m)
                plsc.store_scatter(cy2, [pos], cy2[pl.ds(v * 16, 16)], mask=m)
                plsc.store_scatter(ca, [pos], ca[pl.ds(v * 16, 16)], mask=m)
                plsc.store_scatter(cs, [pos], cs[pl.ds(v * 16, 16)], mask=m)
                plsc.store_scatter(cpos, [pos], cpos[pl.ds(v * 16, 16)], mask=m)
                return wp + jnp.sum(mi)
            wp = lax.fori_loop(0, nv, _cmp2, jnp.int32(0))

            def _wipe(v, _):
                l16 = v * 16 + i16
                live = l16 < wp
                cal[pl.ds(v * 16, 16)] = live.astype(jnp.int32)
                cpos[pl.ds(v * 16, 16)] = jnp.where(
                    live, cpos[pl.ds(v * 16, 16)], BIGc())
                return 0
            lax.fori_loop(0, SB // 16, _wipe, 0)
            kv[pl.ds(32, 16)] = _splat(wp)
        return kv[pl.ds(32, 16)][0]

    mc = lax.fori_loop(0, NBLK, _block, mc0)
    del mc
    plsc.subcore_barrier()

    @pl.when(w == 0)
    def _emit():
        pltpu.sync_copy(sp_kcnt, kv.at[pl.ds(0, 32)])

        def _zero(v, _):
            det[pl.ds(v * 16, 16)] = jnp.zeros((16,), jnp.float32)
            return 0
        lax.fori_loop(0, det.shape[0] // 16, _zero, 0)

        def _blk(b, off):
            chv = (b >> 4) * 16
            kb = kv[pl.ds(chv, 16)][_splat(b - chv)][0]

            @pl.when(kb > 0)
            def _():
                pltpu.sync_copy(sp_kx1.at[pl.ds(b * 256, 256)], pv.at[0])
                pltpu.sync_copy(sp_ky1.at[pl.ds(b * 256, 256)], pv.at[1])
                pltpu.sync_copy(sp_kx2.at[pl.ds(b * 256, 256)], pv.at[2])
                pltpu.sync_copy(sp_ky2.at[pl.ds(b * 256, 256)], pv.at[3])
                pltpu.sync_copy(sp_ks.at[pl.ds(b * 256, 256)], pbs)

                def _row(v, _):
                    l16 = v * 16 + i16
                    ridx = off + l16
                    m = (l16 < kb) & (ridx < jnp.int32(KEEP))
                    base5 = ridx * 5
                    plsc.store_scatter(det, [base5], pv[0, pl.ds(v * 16, 16)], mask=m)
                    plsc.store_scatter(det, [base5 + 1], pv[1, pl.ds(v * 16, 16)], mask=m)
                    plsc.store_scatter(det, [base5 + 2], pv[2, pl.ds(v * 16, 16)], mask=m)
                    plsc.store_scatter(det, [base5 + 3], pv[3, pl.ds(v * 16, 16)], mask=m)
                    plsc.store_scatter(det, [base5 + 4], pbs[pl.ds(v * 16, 16)], mask=m)
                    return 0
                lax.fori_loop(0, BLK // 16, _row, 0)
            return off + kb
        lax.fori_loop(0, NBLK, _blk, jnp.int32(0))
        pltpu.sync_copy(det.at[pl.ds(0, 3760)], out_hbm)


def kernel(boxes, scores):
    scores_p = jnp.zeros((NPAD,), jnp.float32).at[:N].set(scores)
    boxes_t = jnp.zeros((4, NPAD), jnp.float32).at[:, :N].set(boxes.T)
    b1, b2, b3, b4 = boxes_t[0], boxes_t[1], boxes_t[2], boxes_t[3]

    mesh = plsc.VectorSubcoreMesh(
        core_axis_name="c", subcore_axis_name="s", num_cores=1)

    f = pl.kernel(
        _body,
        out_type=jax.ShapeDtypeStruct((3760,), jnp.float32),
        mesh=mesh,
        compiler_params=pltpu.CompilerParams(needs_layout_passes=False),
        scratch_types=[
            pltpu.VMEM_SHARED((NPAD,), jnp.int32),
            pltpu.VMEM_SHARED((NPAD,), jnp.int32),
            pltpu.VMEM_SHARED((NPAD,), jnp.int32),
            pltpu.VMEM_SHARED((NPAD,), jnp.int32),
            pltpu.VMEM_SHARED((NTILES, 256), jnp.int32),
            pltpu.VMEM_SHARED((5, 256), jnp.float32),
            pltpu.VMEM_SHARED((NBLK * 256,), jnp.float32),
            pltpu.VMEM_SHARED((NBLK * 256,), jnp.float32),
            pltpu.VMEM_SHARED((NBLK * 256,), jnp.float32),
            pltpu.VMEM_SHARED((NBLK * 256,), jnp.float32),
            pltpu.VMEM_SHARED((NBLK * 256,), jnp.float32),
            pltpu.VMEM_SHARED((32,), jnp.int32),
            pltpu.VMEM((CH,), jnp.float32),
            pltpu.VMEM((CH,), jnp.int32),
            pltpu.VMEM((CH,), jnp.int32),
            pltpu.VMEM((CH // 128, 128), jnp.int32),
            pltpu.VMEM((NTILES, 256), jnp.int32),
            pltpu.VMEM((256,), jnp.int32),
            pltpu.VMEM((256,), jnp.int32),
            pltpu.VMEM((256,), jnp.int32),
            pltpu.VMEM((NPAD,), jnp.float32),
            pltpu.VMEM((NPAD,), jnp.float32),
            pltpu.VMEM((NPAD,), jnp.float32),
            pltpu.VMEM((NPAD,), jnp.float32),
            pltpu.VMEM((SB + 16,), jnp.float32),
            pltpu.VMEM((SB + 16,), jnp.float32),
            pltpu.VMEM((SB + 16,), jnp.float32),
            pltpu.VMEM((SB + 16,), jnp.float32),
            pltpu.VMEM((SB + 16,), jnp.float32),
            pltpu.VMEM((SB + 16,), jnp.float32),
            pltpu.VMEM((SB + 16,), jnp.int32),
            pltpu.VMEM((SB + 16,), jnp.int32),
            pltpu.VMEM((SB + 16,), jnp.int32),
            pltpu.VMEM((5, 256), jnp.float32),
            pltpu.VMEM((5, 256), jnp.float32),
            pltpu.VMEM((256,), jnp.float32),
            pltpu.VMEM((48,), jnp.int32),
            pltpu.VMEM((3760,), jnp.float32),
        ],
    )
    out = f(scores_p, b1, b2, b3, b4)
    return out[:3750].reshape(KEEP, 5)
